# Initial kernel scaffold; baseline (speedup 1.0000x reference)
import jax
import jax.numpy as jnp
from jax.experimental import pallas as pl


def kernel(edge_index, edge_weight, adj, embedding, W_item, W_i1, W_i2, channel):
    raise NotImplementedError("write your pallas kernel here")


# speedup vs baseline: 2.1237x; 2.1237x over previous
import functools

import jax
import jax.numpy as jnp
from jax import lax
from jax.experimental import pallas as pl
from jax.experimental.pallas import tpu as pltpu
from jax.experimental.pallas import tpu_sc as plsc

N = 50000
E = 800000
EMB = 100
K = 100
LAYERS = 3

D = 128
R = 392
NB = 128
NP = NB * R
TN = 512
GRID = NP // TN
CH = 256
EPAD = E + CH
NC_ = 2
NS_ = 16
BPW = NB // (NC_ * NS_)
NEG = -1e30


def _spmm_body(x_hbm, src_hbm, ew_hbm, dst_hbm, starts_hbm, out_hbm,
               starts_v, src_v, ew_v, dst_v, rows_v, acc_v, sem):
    wid = lax.axis_index("s") * NC_ + lax.axis_index("c")
    pltpu.sync_copy(starts_hbm, starts_v)
    lane = lax.iota(jnp.int32, 16)
    for p in range(BPW):
        b = wid * BPW + p
        row_base = b * R
        sv = starts_v[pl.ds(b, 16)]
        e0 = sv[0]
        e1 = sv[1]

        def zbody(j, _):
            acc_v[pl.ds(j * 16, 16)] = jnp.zeros((16,), jnp.float32)
            return 0
        lax.fori_loop(0, R * D // 16, zbody, 0)

        ca0 = (e0 // 8) * 8
        nch = (e1 - ca0 + CH - 1) // CH

        def chunk_body(j, _):
            ca = ca0 + j * CH
            pltpu.sync_copy(src_hbm.at[pl.ds(ca, CH)], src_v)
            pltpu.sync_copy(ew_hbm.at[pl.ds(ca, CH)], ew_v)
            pltpu.sync_copy(dst_hbm.at[pl.ds(ca, CH)], dst_v)
            pltpu.async_copy(x_hbm.at[src_v], rows_v, sem).wait()
            lo = jnp.maximum(e0, ca) - ca
            hi = jnp.minimum(e1, ca + CH) - ca

            def group_body(g, _):
                i0 = g * 16
                dlv = jnp.clip(dst_v[pl.ds(i0, 16)] - row_base, 0, R - 1)
                ok = (i0 + lane >= lo) & (i0 + lane < hi)
                wvec = jnp.where(ok, ew_v[pl.ds(i0, 16)], 0.0)
                for j in range(16):
                    base = dlv[j] * D
                    wv = jnp.full((16,), wvec[j], jnp.float32)
                    for k in range(EMB // 16 + 1):
                        off = base + k * 16
                        acc_v[pl.ds(off, 16)] = (
                            acc_v[pl.ds(off, 16)]
                            + wv * rows_v[i0 + j, pl.ds(k * 16, 16)])
                return 0
            lax.fori_loop(lo // 16, (hi + 15) // 16, group_body, 0)
            return 0
        lax.fori_loop(0, nch, chunk_body, 0)
        pltpu.sync_copy(acc_v, out_hbm.at[pl.ds(row_base * D, R * D)])


def _sc_spmm(x, src, ew, dst, starts):
    f = pl.kernel(
        _spmm_body,
        out_type=jax.ShapeDtypeStruct((NP * D,), jnp.float32),
        mesh=plsc.VectorSubcoreMesh(core_axis_name="c", subcore_axis_name="s"),
        scratch_types=[
            pltpu.VMEM((NB + 16,), jnp.int32),
            pltpu.VMEM((CH,), jnp.int32),
            pltpu.VMEM((CH,), jnp.float32),
            pltpu.VMEM((CH,), jnp.int32),
            pltpu.VMEM((CH, D), jnp.float32),
            pltpu.VMEM((R * D,), jnp.float32),
            pltpu.SemaphoreType.DMA,
        ],
    )
    return f(x, src, ew, dst, starts).reshape(NP, D)


def _lin_body(x_ref, w_ref, o_ref):
    o_ref[...] = jnp.dot(x_ref[...], w_ref[...],
                         preferred_element_type=jnp.float32)


def _tc_linear(x, wt):
    return pl.pallas_call(
        _lin_body,
        grid=(GRID,),
        in_specs=[pl.BlockSpec((TN, D), lambda i: (i, 0)),
                  pl.BlockSpec((D, D), lambda i: (0, 0))],
        out_specs=pl.BlockSpec((TN, D), lambda i: (i, 0)),
        out_shape=jax.ShapeDtypeStruct((NP, D), jnp.float32),
    )(x, wt)


def _cluster_body(agg_ref, agg2_ref, adj_ref, w1_ref, w2_ref, p_ref, hk_ref):
    a = agg_ref[...]
    y1 = jnp.dot(a, w1_ref[...], preferred_element_type=jnp.float32)
    h1 = jnp.maximum(y1 + agg2_ref[...], 0.0)
    y2 = jnp.dot(h1, w2_ref[...], preferred_element_type=jnp.float32)
    col = lax.broadcasted_iota(jnp.int32, (TN, D), 1)
    logits = jnp.where(col < K, y2, NEG)
    m = jnp.max(logits, axis=1, keepdims=True)
    ex = jnp.exp(logits - m)
    p = ex / jnp.sum(ex, axis=1, keepdims=True)
    p_ref[...] = p
    adjv = adj_ref[...]
    denom = jnp.sum(p * adjv, axis=1, keepdims=True)
    aw = a * (adjv / denom)
    contrib = lax.dot_general(p, aw, (((0,), (0,)), ((), ())),
                              preferred_element_type=jnp.float32)

    @pl.when(pl.program_id(0) == 0)
    def _():
        hk_ref[...] = jnp.zeros_like(hk_ref)
    hk_ref[...] += contrib


def _tc_cluster(agg, adj2, w1t, w2t):
    return pl.pallas_call(
        _cluster_body,
        grid=(GRID,),
        in_specs=[pl.BlockSpec((TN, D), lambda i: (i, 0)),
                  pl.BlockSpec((TN, D), lambda i: (i, 0)),
                  pl.BlockSpec((TN, 1), lambda i: (i, 0)),
                  pl.BlockSpec((D, D), lambda i: (0, 0)),
                  pl.BlockSpec((D, D), lambda i: (0, 0))],
        out_specs=[pl.BlockSpec((TN, D), lambda i: (i, 0)),
                   pl.BlockSpec((D, D), lambda i: (0, 0))],
        out_shape=[jax.ShapeDtypeStruct((NP, D), jnp.float32),
                   jax.ShapeDtypeStruct((D, D), jnp.float32)],
    )(agg, agg, adj2, w1t, w2t)


def _norm_rows(x):
    n = jnp.sqrt(jnp.sum(x * x, axis=1, keepdims=True))
    return x / jnp.maximum(n, 1e-12)


def _pool_body(p_ref, hk_ref, agg_ref, fin_ref, finh_ref, wn_ref,
               xlin_ref, fout_ref, fhout_ref):
    hb = jnp.dot(p_ref[...], hk_ref[...], preferred_element_type=jnp.float32)
    xn = hb + agg_ref[...]
    fout_ref[...] = fin_ref[...] + _norm_rows(xn)
    fhout_ref[...] = finh_ref[...] + _norm_rows(hb)
    xlin_ref[...] = jnp.dot(xn, wn_ref[...], preferred_element_type=jnp.float32)


def _tc_pool(p, hk, agg, fin, finh, wnt):
    return pl.pallas_call(
        _pool_body,
        grid=(GRID,),
        in_specs=[pl.BlockSpec((TN, D), lambda i: (i, 0)),
                  pl.BlockSpec((D, D), lambda i: (0, 0)),
                  pl.BlockSpec((TN, D), lambda i: (i, 0)),
                  pl.BlockSpec((TN, D), lambda i: (i, 0)),
                  pl.BlockSpec((TN, D), lambda i: (i, 0)),
                  pl.BlockSpec((D, D), lambda i: (0, 0))],
        out_specs=[pl.BlockSpec((TN, D), lambda i: (i, 0)),
                   pl.BlockSpec((TN, D), lambda i: (i, 0)),
                   pl.BlockSpec((TN, D), lambda i: (i, 0))],
        out_shape=[jax.ShapeDtypeStruct((NP, D), jnp.float32),
                   jax.ShapeDtypeStruct((NP, D), jnp.float32),
                   jax.ShapeDtypeStruct((NP, D), jnp.float32)],
    )(p, hk, agg, fin, finh, wnt)


def _pool_last_body(p_ref, hk_ref, agg_ref, fin_ref, finh_ref,
                    item_ref, hs_ref):
    hb = jnp.dot(p_ref[...], hk_ref[...], preferred_element_type=jnp.float32)
    xn = hb + agg_ref[...]
    item_ref[...] = (fin_ref[...] + _norm_rows(xn)) / (LAYERS + 1)
    hs_ref[...] = (finh_ref[...] + _norm_rows(hb)) / LAYERS


def _tc_pool_last(p, hk, agg, fin, finh):
    return pl.pallas_call(
        _pool_last_body,
        grid=(GRID,),
        in_specs=[pl.BlockSpec((TN, D), lambda i: (i, 0)),
                  pl.BlockSpec((D, D), lambda i: (0, 0)),
                  pl.BlockSpec((TN, D), lambda i: (i, 0)),
                  pl.BlockSpec((TN, D), lambda i: (i, 0)),
                  pl.BlockSpec((TN, D), lambda i: (i, 0))],
        out_specs=[pl.BlockSpec((TN, D), lambda i: (i, 0)),
                   pl.BlockSpec((TN, D), lambda i: (i, 0))],
        out_shape=[jax.ShapeDtypeStruct((NP, D), jnp.float32),
                   jax.ShapeDtypeStruct((NP, D), jnp.float32)],
    )(p, hk, agg, fin, finh)


def kernel(edge_index, edge_weight, adj, embedding, W_item, W_i1, W_i2,
           channel):
    del channel
    f32 = jnp.float32
    dst = edge_index[0].astype(jnp.int32)
    src = edge_index[1].astype(jnp.int32)
    ew = edge_weight.astype(f32)

    sdst, ssrc, sew = lax.sort((dst, src, ew), num_keys=1)
    bounds = (jnp.arange(NB + 1, dtype=jnp.int32) * R).astype(jnp.int32)
    starts = jnp.searchsorted(sdst, bounds).astype(jnp.int32)
    starts = jnp.concatenate([starts, jnp.full((15,), E, jnp.int32)])
    ssrc = jnp.concatenate([ssrc, jnp.zeros((CH,), jnp.int32)])
    sew = jnp.concatenate([sew, jnp.zeros((CH,), f32)])
    sdst = jnp.concatenate([sdst, jnp.full((CH,), NP - 1, jnp.int32)])

    emb_pad = jnp.zeros((NP, D), f32).at[:N, :EMB].set(embedding)
    adj2 = jnp.ones((NP, 1), f32).at[:N, 0].set(adj)
    wt = jnp.zeros((LAYERS, D, D), f32).at[:, :EMB, :EMB].set(
        jnp.transpose(W_item, (0, 2, 1)))
    w1t = jnp.zeros((D, D), f32).at[:EMB, :EMB].set(W_i1.T)
    w2t = jnp.zeros((D, D), f32).at[:EMB, :K].set(W_i2.T)

    fin = emb_pad
    finh = jnp.zeros((NP, D), f32)
    xlin = _tc_linear(emb_pad, wt[0])
    for i in range(LAYERS):
        agg = _sc_spmm(xlin, ssrc, sew, sdst, starts)
        p, hk = _tc_cluster(agg, adj2, w1t, w2t)
        if i + 1 < LAYERS:
            xlin, fin, finh = _tc_pool(p, hk, agg, fin, finh, wt[i + 1])
        else:
            item, hs = _tc_pool_last(p, hk, agg, fin, finh)
    return item[:N, :EMB], hs[:N, :EMB]


# speedup vs baseline: 2.2198x; 1.0452x over previous
import functools

import jax
import jax.numpy as jnp
from jax import lax
from jax.experimental import pallas as pl
from jax.experimental.pallas import tpu as pltpu
from jax.experimental.pallas import tpu_sc as plsc

N = 50000
E = 800000
EMB = 100
K = 100
LAYERS = 3

D = 128
R = 392
NB = 128
NP = NB * R
TN = 512
GRID = NP // TN
CH = 512
EPAD = E + CH
NC_ = 2
NS_ = 16
BPW = NB // (NC_ * NS_)
NEG = -1e30


def _spmm_body(x_hbm, src_hbm, ew_hbm, dst_hbm, starts_hbm, out_hbm,
               starts_v, src_v, ew_v, dst_v, rows_v, acc_v, sem):
    wid = lax.axis_index("s") * NC_ + lax.axis_index("c")
    pltpu.sync_copy(starts_hbm, starts_v)
    lane = lax.iota(jnp.int32, 16)
    for p in range(BPW):
        b = wid * BPW + p
        row_base = b * R
        sv = starts_v[pl.ds(b, 16)]
        e0 = sv[0]
        e1 = sv[1]

        def zbody(j, _):
            acc_v[pl.ds(j * 16, 16)] = jnp.zeros((16,), jnp.float32)
            return 0
        lax.fori_loop(0, R * D // 16, zbody, 0)

        ca0 = (e0 // 8) * 8
        nch = (e1 - ca0 + CH - 1) // CH

        def chunk_body(j, _):
            ca = ca0 + j * CH
            pltpu.sync_copy(src_hbm.at[pl.ds(ca, CH)], src_v)
            pltpu.sync_copy(ew_hbm.at[pl.ds(ca, CH)], ew_v)
            pltpu.sync_copy(dst_hbm.at[pl.ds(ca, CH)], dst_v)
            pltpu.async_copy(x_hbm.at[src_v], rows_v, sem).wait()
            lo = jnp.maximum(e0, ca) - ca
            hi = jnp.minimum(e1, ca + CH) - ca

            def group_body(g, _):
                i0 = g * 16
                dlv = jnp.clip(dst_v[pl.ds(i0, 16)] - row_base, 0, R - 1)
                ok = (i0 + lane >= lo) & (i0 + lane < hi)
                wvec = jnp.where(ok, ew_v[pl.ds(i0, 16)], 0.0)
                for j in range(16):
                    base = dlv[j] * D
                    wv = jnp.full((16,), wvec[j], jnp.float32)
                    for k in range(EMB // 16 + 1):
                        off = base + k * 16
                        acc_v[pl.ds(off, 16)] = (
                            acc_v[pl.ds(off, 16)]
                            + wv * rows_v[i0 + j, pl.ds(k * 16, 16)])
                return 0
            lax.fori_loop(lo // 16, (hi + 15) // 16, group_body, 0)
            return 0
        lax.fori_loop(0, nch, chunk_body, 0)
        pltpu.sync_copy(acc_v, out_hbm.at[pl.ds(row_base * D, R * D)])


def _sc_spmm(x, src, ew, dst, starts):
    f = pl.kernel(
        _spmm_body,
        out_type=jax.ShapeDtypeStruct((NP * D,), jnp.float32),
        mesh=plsc.VectorSubcoreMesh(core_axis_name="c", subcore_axis_name="s"),
        scratch_types=[
            pltpu.VMEM((NB + 16,), jnp.int32),
            pltpu.VMEM((CH,), jnp.int32),
            pltpu.VMEM((CH,), jnp.float32),
            pltpu.VMEM((CH,), jnp.int32),
            pltpu.VMEM((CH, D), jnp.float32),
            pltpu.VMEM((R * D,), jnp.float32),
            pltpu.SemaphoreType.DMA,
        ],
    )
    return f(x, src, ew, dst, starts).reshape(NP, D)


def _lin_body(x_ref, w_ref, o_ref):
    o_ref[...] = jnp.dot(x_ref[...], w_ref[...],
                         preferred_element_type=jnp.float32)


def _tc_linear(x, wt):
    return pl.pallas_call(
        _lin_body,
        grid=(GRID,),
        in_specs=[pl.BlockSpec((TN, D), lambda i: (i, 0)),
                  pl.BlockSpec((D, D), lambda i: (0, 0))],
        out_specs=pl.BlockSpec((TN, D), lambda i: (i, 0)),
        out_shape=jax.ShapeDtypeStruct((NP, D), jnp.float32),
    )(x, wt)


def _cluster_body(agg_ref, agg2_ref, adj_ref, w1_ref, w2_ref, p_ref, hk_ref):
    a = agg_ref[...]
    y1 = jnp.dot(a, w1_ref[...], preferred_element_type=jnp.float32)
    h1 = jnp.maximum(y1 + agg2_ref[...], 0.0)
    y2 = jnp.dot(h1, w2_ref[...], preferred_element_type=jnp.float32)
    col = lax.broadcasted_iota(jnp.int32, (TN, D), 1)
    logits = jnp.where(col < K, y2, NEG)
    m = jnp.max(logits, axis=1, keepdims=True)
    ex = jnp.exp(logits - m)
    p = ex / jnp.sum(ex, axis=1, keepdims=True)
    p_ref[...] = p
    adjv = adj_ref[...]
    denom = jnp.sum(p * adjv, axis=1, keepdims=True)
    aw = a * (adjv / denom)
    contrib = lax.dot_general(p, aw, (((0,), (0,)), ((), ())),
                              preferred_element_type=jnp.float32)

    @pl.when(pl.program_id(0) == 0)
    def _():
        hk_ref[...] = jnp.zeros_like(hk_ref)
    hk_ref[...] += contrib


def _tc_cluster(agg, adj2, w1t, w2t):
    return pl.pallas_call(
        _cluster_body,
        grid=(GRID,),
        in_specs=[pl.BlockSpec((TN, D), lambda i: (i, 0)),
                  pl.BlockSpec((TN, D), lambda i: (i, 0)),
                  pl.BlockSpec((TN, 1), lambda i: (i, 0)),
                  pl.BlockSpec((D, D), lambda i: (0, 0)),
                  pl.BlockSpec((D, D), lambda i: (0, 0))],
        out_specs=[pl.BlockSpec((TN, D), lambda i: (i, 0)),
                   pl.BlockSpec((D, D), lambda i: (0, 0))],
        out_shape=[jax.ShapeDtypeStruct((NP, D), jnp.float32),
                   jax.ShapeDtypeStruct((D, D), jnp.float32)],
    )(agg, agg, adj2, w1t, w2t)


def _norm_rows(x):
    n = jnp.sqrt(jnp.sum(x * x, axis=1, keepdims=True))
    return x / jnp.maximum(n, 1e-12)


def _pool_body(p_ref, hk_ref, agg_ref, fin_ref, finh_ref, wn_ref,
               xlin_ref, fout_ref, fhout_ref):
    hb = jnp.dot(p_ref[...], hk_ref[...], preferred_element_type=jnp.float32)
    xn = hb + agg_ref[...]
    fout_ref[...] = fin_ref[...] + _norm_rows(xn)
    fhout_ref[...] = finh_ref[...] + _norm_rows(hb)
    xlin_ref[...] = jnp.dot(xn, wn_ref[...], preferred_element_type=jnp.float32)


def _tc_pool(p, hk, agg, fin, finh, wnt):
    return pl.pallas_call(
        _pool_body,
        grid=(GRID,),
        in_specs=[pl.BlockSpec((TN, D), lambda i: (i, 0)),
                  pl.BlockSpec((D, D), lambda i: (0, 0)),
                  pl.BlockSpec((TN, D), lambda i: (i, 0)),
                  pl.BlockSpec((TN, D), lambda i: (i, 0)),
                  pl.BlockSpec((TN, D), lambda i: (i, 0)),
                  pl.BlockSpec((D, D), lambda i: (0, 0))],
        out_specs=[pl.BlockSpec((TN, D), lambda i: (i, 0)),
                   pl.BlockSpec((TN, D), lambda i: (i, 0)),
                   pl.BlockSpec((TN, D), lambda i: (i, 0))],
        out_shape=[jax.ShapeDtypeStruct((NP, D), jnp.float32),
                   jax.ShapeDtypeStruct((NP, D), jnp.float32),
                   jax.ShapeDtypeStruct((NP, D), jnp.float32)],
    )(p, hk, agg, fin, finh, wnt)


def _pool_last_body(p_ref, hk_ref, agg_ref, fin_ref, finh_ref,
                    item_ref, hs_ref):
    hb = jnp.dot(p_ref[...], hk_ref[...], preferred_element_type=jnp.float32)
    xn = hb + agg_ref[...]
    item_ref[...] = (fin_ref[...] + _norm_rows(xn)) / (LAYERS + 1)
    hs_ref[...] = (finh_ref[...] + _norm_rows(hb)) / LAYERS


def _tc_pool_last(p, hk, agg, fin, finh):
    return pl.pallas_call(
        _pool_last_body,
        grid=(GRID,),
        in_specs=[pl.BlockSpec((TN, D), lambda i: (i, 0)),
                  pl.BlockSpec((D, D), lambda i: (0, 0)),
                  pl.BlockSpec((TN, D), lambda i: (i, 0)),
                  pl.BlockSpec((TN, D), lambda i: (i, 0)),
                  pl.BlockSpec((TN, D), lambda i: (i, 0))],
        out_specs=[pl.BlockSpec((TN, D), lambda i: (i, 0)),
                   pl.BlockSpec((TN, D), lambda i: (i, 0))],
        out_shape=[jax.ShapeDtypeStruct((NP, D), jnp.float32),
                   jax.ShapeDtypeStruct((NP, D), jnp.float32)],
    )(p, hk, agg, fin, finh)


def kernel(edge_index, edge_weight, adj, embedding, W_item, W_i1, W_i2,
           channel):
    del channel
    f32 = jnp.float32
    dst = edge_index[0].astype(jnp.int32)
    src = edge_index[1].astype(jnp.int32)
    ew = edge_weight.astype(f32)

    sdst, ssrc, sew = lax.sort((dst, src, ew), num_keys=1)
    bounds = (jnp.arange(NB + 1, dtype=jnp.int32) * R).astype(jnp.int32)
    starts = jnp.searchsorted(sdst, bounds).astype(jnp.int32)
    starts = jnp.concatenate([starts, jnp.full((15,), E, jnp.int32)])
    ssrc = jnp.concatenate([ssrc, jnp.zeros((CH,), jnp.int32)])
    sew = jnp.concatenate([sew, jnp.zeros((CH,), f32)])
    sdst = jnp.concatenate([sdst, jnp.full((CH,), NP - 1, jnp.int32)])

    emb_pad = jnp.zeros((NP, D), f32).at[:N, :EMB].set(embedding)
    adj2 = jnp.ones((NP, 1), f32).at[:N, 0].set(adj)
    wt = jnp.zeros((LAYERS, D, D), f32).at[:, :EMB, :EMB].set(
        jnp.transpose(W_item, (0, 2, 1)))
    w1t = jnp.zeros((D, D), f32).at[:EMB, :EMB].set(W_i1.T)
    w2t = jnp.zeros((D, D), f32).at[:EMB, :K].set(W_i2.T)

    fin = emb_pad
    finh = jnp.zeros((NP, D), f32)
    xlin = _tc_linear(emb_pad, wt[0])
    for i in range(LAYERS):
        agg = _sc_spmm(xlin, ssrc, sew, sdst, starts)
        p, hk = _tc_cluster(agg, adj2, w1t, w2t)
        if i + 1 < LAYERS:
            xlin, fin, finh = _tc_pool(p, hk, agg, fin, finh, wt[i + 1])
        else:
            item, hs = _tc_pool_last(p, hk, agg, fin, finh)
    return item[:N, :EMB], hs[:N, :EMB]
